# baseline (speedup 1.0000x reference)
import functools

import jax
import jax.numpy as jnp
from jax import lax
from jax.experimental import pallas as pl
from jax.experimental.pallas import tpu as pltpu
from jax.experimental.pallas import tpu_sc as plsc

_MASK_PCT = 0.8
_CAP = 112
_CHUNK = 128
_NBUF = 4


def _make_field_idx(bs: int):
    n_masked = int(_MASK_PCT * bs)
    jkey = jax.random.key(42)
    j = jax.random.randint(jkey, (n_masked,), 0, 5, dtype=jnp.int32)
    return jnp.concatenate([j, -jnp.ones((bs - n_masked,), dtype=jnp.int32)])


def _scatter_indices(j, B):
    half = B // 2
    jh = j.reshape(2, half)
    base = jnp.array([[0], [half]], dtype=jnp.int32)
    lists = []
    for f in range(3):
        m = jh == f
        order = jnp.argsort(~m, axis=1, stable=True)[:, :16 * _CAP]
        glob = order.astype(jnp.int32) + base
        cnt = m.sum(axis=1).astype(jnp.int32)
        k_i = jnp.arange(16 * _CAP, dtype=jnp.int32)
        lst = jnp.where(k_i[None, :] < cnt[:, None], glob, glob[:, :1])
        lists.append(lst.reshape(2, 16, _CAP))
    idx = jnp.stack(lists, axis=2)
    return idx.reshape(32, 3, _CAP)


def _sc_mask_copy(dgb, prb, odb, dgp, prp, odp, idx_hbm, zeros_hbm,
                  o_dgb, o_prb, o_odb, o_dgp, o_prp, o_odp,
                  bufs0, bufs1, bufs2, bufs3, zbuf, idxb,
                  sem_in0, sem_in1, sem_in2, sem_in3,
                  sem_out0, sem_out1, sem_out2, sem_out3, sem_sc):
    bufs = (bufs0, bufs1, bufs2, bufs3)
    sem_in = (sem_in0, sem_in1, sem_in2, sem_in3)
    sem_out = (sem_out0, sem_out1, sem_out2, sem_out3)
    c = lax.axis_index("c")
    s = lax.axis_index("s")
    wid = c * 16 + s
    row0 = c * 8192 + s * 512

    pltpu.sync_copy(idx_hbm.at[wid], idxb)
    pltpu.sync_copy(zeros_hbm, zbuf)

    pairs = [(dgb, o_dgb), (prb, o_prb), (odb, o_odb),
             (dgp, o_dgp), (prp, o_prp), (odp, o_odp)]
    chunks = []
    for src, dst in pairs:
        for cchunk in range(512 // _CHUNK):
            sl = pl.ds(row0 + cchunk * _CHUNK, _CHUNK)
            chunks.append((src.at[sl], dst.at[sl]))

    n = len(chunks)
    in_h = [None] * n
    out_h = [None] * n
    for k in range(min(_NBUF, n)):
        in_h[k] = pltpu.async_copy(chunks[k][0], bufs[k % _NBUF],
                                   sem_in[k % _NBUF])
    for k in range(n):
        in_h[k].wait()
        out_h[k] = pltpu.async_copy(bufs[k % _NBUF], chunks[k][1],
                                    sem_out[k % _NBUF])
        kn = k + _NBUF
        if kn < n:
            out_h[k].wait()
            in_h[kn] = pltpu.async_copy(chunks[kn][0], bufs[kn % _NBUF],
                                        sem_in[kn % _NBUF])
    for k in range(max(0, n - _NBUF), n):
        out_h[k].wait()

    plsc.subcore_barrier()

    outs_by_field = [(o_dgb, o_dgp), (o_prb, o_prp), (o_odb, o_odp)]
    sc_h = []
    for f in range(3):
        for dst in outs_by_field[f]:
            sc_h.append(pltpu.async_copy(zbuf, dst.at[idxb.at[f]], sem_sc))
    for h in sc_h:
        h.wait()


def _vec_mask_kernel(j_ref, age_ref, gen_ref, o_age, o_gen):
    j = j_ref[...]
    z = jnp.float32(0.0)
    o_age[...] = jnp.where(j != 3, age_ref[...], z)
    o_gen[...] = jnp.where(j != 4, gen_ref[...], z)


def kernel(x_dg_bin, x_prod_bin, x_odb_bin, x_dg_pe, x_prod_pe, x_odb_pe,
           x_age, x_gender):
    B, D = x_dg_bin.shape
    R = B // 128
    j = _make_field_idx(B)
    idx = _scatter_indices(j, B)
    zeros = jnp.zeros((_CAP, D), jnp.float32)

    big_t = jax.ShapeDtypeStruct((B, D), jnp.float32)
    mesh = plsc.VectorSubcoreMesh(core_axis_name="c", subcore_axis_name="s")
    sc_fn = pl.kernel(
        _sc_mask_copy,
        out_type=[big_t] * 6,
        mesh=mesh,
        scratch_types=(
            [pltpu.VMEM((_CHUNK, D), jnp.float32)] * _NBUF
            + [pltpu.VMEM((_CAP, D), jnp.float32),
               pltpu.VMEM((3, _CAP), jnp.int32)]
            + [pltpu.SemaphoreType.DMA] * 9
        ),
    )
    o_dgb, o_prb, o_odb, o_dgp, o_prp, o_odp = sc_fn(
        x_dg_bin, x_prod_bin, x_odb_bin, x_dg_pe, x_prod_pe, x_odb_pe,
        idx, zeros)

    jp = j.reshape(R, 128)
    vec = pl.BlockSpec((R, 128), lambda: (0, 0))
    vec_t = jax.ShapeDtypeStruct((R, 128), jnp.float32)
    o_age, o_gen = pl.pallas_call(
        _vec_mask_kernel,
        in_specs=[vec, vec, vec],
        out_specs=[vec, vec],
        out_shape=[vec_t, vec_t],
    )(jp, x_age.reshape(R, 128), x_gender.reshape(R, 128))

    return (o_dgb, o_prb, o_odb, o_dgp, o_prp, o_odp,
            o_age.reshape(B), o_gen.reshape(B))


# speedup vs baseline: 1.0211x; 1.0211x over previous
import functools

import jax
import jax.numpy as jnp
from jax import lax
from jax.experimental import pallas as pl
from jax.experimental.pallas import tpu as pltpu
from jax.experimental.pallas import tpu_sc as plsc

_MASK_PCT = 0.8
_CAP = 112
_CHUNK = 64
_NBUF = 8


def _make_field_idx(bs: int):
    n_masked = int(_MASK_PCT * bs)
    jkey = jax.random.key(42)
    j = jax.random.randint(jkey, (n_masked,), 0, 5, dtype=jnp.int32)
    return jnp.concatenate([j, -jnp.ones((bs - n_masked,), dtype=jnp.int32)])


def _scatter_indices(j, B):
    half = B // 2
    jh = j.reshape(2, half)
    base = jnp.array([[0], [half]], dtype=jnp.int32)
    lists = []
    for f in range(3):
        m = jh == f
        order = jnp.argsort(~m, axis=1, stable=True)[:, :16 * _CAP]
        glob = order.astype(jnp.int32) + base
        cnt = m.sum(axis=1).astype(jnp.int32)
        k_i = jnp.arange(16 * _CAP, dtype=jnp.int32)
        lst = jnp.where(k_i[None, :] < cnt[:, None], glob, glob[:, :1])
        lists.append(lst.reshape(2, 16, _CAP))
    idx = jnp.stack(lists, axis=2)
    return idx.reshape(32, 3, _CAP)


def _sc_mask_copy(*refs):
    (dgb, prb, odb, dgp, prp, odp, idx_hbm, zeros_hbm,
     o_dgb, o_prb, o_odb, o_dgp, o_prp, o_odp) = refs[:14]
    bufs = refs[14:14 + _NBUF]
    zbuf, idxb = refs[14 + _NBUF:16 + _NBUF]
    sem_in = refs[16 + _NBUF:16 + 2 * _NBUF]
    sem_out = refs[16 + 2 * _NBUF:16 + 3 * _NBUF]
    sem_sc = refs[16 + 3 * _NBUF]

    c = lax.axis_index("c")
    s = lax.axis_index("s")
    wid = c * 16 + s
    row0 = c * 8192 + s * 512

    pltpu.sync_copy(idx_hbm.at[wid], idxb)
    pltpu.sync_copy(zeros_hbm, zbuf)

    pairs = [(dgb, o_dgb), (prb, o_prb), (odb, o_odb),
             (dgp, o_dgp), (prp, o_prp), (odp, o_odp)]
    chunks = []
    for src, dst in pairs:
        for cchunk in range(512 // _CHUNK):
            sl = pl.ds(row0 + cchunk * _CHUNK, _CHUNK)
            chunks.append((src.at[sl], dst.at[sl]))

    n = len(chunks)
    half = _NBUF // 2
    in_h = [None] * n
    out_h = [None] * n

    def start_in(k):
        return pltpu.async_copy(chunks[k][0], bufs[k % _NBUF],
                                sem_in[k % _NBUF])

    def start_out(k):
        return pltpu.async_copy(bufs[k % _NBUF], chunks[k][1],
                                sem_out[k % _NBUF])

    for k in range(min(half, n)):
        in_h[k] = start_in(k)
    for k in range(n):
        in_h[k].wait()
        out_h[k] = start_out(k)
        kn = k + half
        if kn < n:
            ko = kn - _NBUF
            if ko >= 0:
                out_h[ko].wait()
            in_h[kn] = start_in(kn)
    for k in range(max(0, n - _NBUF), n):
        out_h[k].wait()

    plsc.subcore_barrier()

    outs_by_field = [(o_dgb, o_dgp), (o_prb, o_prp), (o_odb, o_odp)]
    sc_h = []
    for f in range(3):
        for dst in outs_by_field[f]:
            sc_h.append(pltpu.async_copy(zbuf, dst.at[idxb.at[f]], sem_sc))
    for h in sc_h:
        h.wait()


def _vec_mask_kernel(j_ref, age_ref, gen_ref, o_age, o_gen):
    j = j_ref[...]
    z = jnp.float32(0.0)
    o_age[...] = jnp.where(j != 3, age_ref[...], z)
    o_gen[...] = jnp.where(j != 4, gen_ref[...], z)


def kernel(x_dg_bin, x_prod_bin, x_odb_bin, x_dg_pe, x_prod_pe, x_odb_pe,
           x_age, x_gender):
    B, D = x_dg_bin.shape
    R = B // 128
    j = _make_field_idx(B)
    idx = _scatter_indices(j, B)
    zeros = jnp.zeros((_CAP, D), jnp.float32)

    big_t = jax.ShapeDtypeStruct((B, D), jnp.float32)
    mesh = plsc.VectorSubcoreMesh(core_axis_name="c", subcore_axis_name="s")
    sc_fn = pl.kernel(
        _sc_mask_copy,
        out_type=[big_t] * 6,
        mesh=mesh,
        scratch_types=(
            [pltpu.VMEM((_CHUNK, D), jnp.float32)] * _NBUF
            + [pltpu.VMEM((_CAP, D), jnp.float32),
               pltpu.VMEM((3, _CAP), jnp.int32)]
            + [pltpu.SemaphoreType.DMA] * (2 * _NBUF + 1)
        ),
    )
    o_dgb, o_prb, o_odb, o_dgp, o_prp, o_odp = sc_fn(
        x_dg_bin, x_prod_bin, x_odb_bin, x_dg_pe, x_prod_pe, x_odb_pe,
        idx, zeros)

    jp = j.reshape(R, 128)
    vec = pl.BlockSpec((R, 128), lambda: (0, 0))
    vec_t = jax.ShapeDtypeStruct((R, 128), jnp.float32)
    o_age, o_gen = pl.pallas_call(
        _vec_mask_kernel,
        in_specs=[vec, vec, vec],
        out_specs=[vec, vec],
        out_shape=[vec_t, vec_t],
    )(jp, x_age.reshape(R, 128), x_gender.reshape(R, 128))

    return (o_dgb, o_prb, o_odb, o_dgp, o_prp, o_odp,
            o_age.reshape(B), o_gen.reshape(B))


# speedup vs baseline: 1.8631x; 1.8245x over previous
import functools

import jax
import jax.numpy as jnp
from jax import lax
from jax.experimental import pallas as pl
from jax.experimental.pallas import tpu as pltpu
from jax.experimental.pallas import tpu_sc as plsc

_MASK_PCT = 0.8
_CAP = 112
_CHUNK = 128
_NBUF = 4
_SC_FIELD = 2


def _make_field_idx(bs: int):
    n_masked = int(_MASK_PCT * bs)
    jkey = jax.random.key(42)
    j = jax.random.randint(jkey, (n_masked,), 0, 5, dtype=jnp.int32)
    return jnp.concatenate([j, -jnp.ones((bs - n_masked,), dtype=jnp.int32)])


def _scatter_indices(j, B, field):
    half = B // 2
    jh = j.reshape(2, half)
    base = jnp.array([[0], [half]], dtype=jnp.int32)
    m = jh == field
    order = jnp.argsort(~m, axis=1, stable=True)[:, :16 * _CAP]
    glob = order.astype(jnp.int32) + base
    cnt = m.sum(axis=1).astype(jnp.int32)
    k_i = jnp.arange(16 * _CAP, dtype=jnp.int32)
    lst = jnp.where(k_i[None, :] < cnt[:, None], glob, glob[:, :1])
    return lst.reshape(32, 1, _CAP)


def _sc_mask_copy(*refs):
    (src_a, src_b, idx_hbm, zeros_hbm, dst_a, dst_b) = refs[:6]
    bufs = refs[6:6 + _NBUF]
    zbuf, idxb = refs[6 + _NBUF:8 + _NBUF]
    sem_in = refs[8 + _NBUF:8 + 2 * _NBUF]
    sem_out = refs[8 + 2 * _NBUF:8 + 3 * _NBUF]
    sem_sc = refs[8 + 3 * _NBUF]

    c = lax.axis_index("c")
    s = lax.axis_index("s")
    wid = c * 16 + s
    row0 = c * 8192 + s * 512

    pltpu.sync_copy(idx_hbm.at[wid], idxb)
    pltpu.sync_copy(zeros_hbm, zbuf)

    chunks = []
    for src, dst in ((src_a, dst_a), (src_b, dst_b)):
        for cchunk in range(512 // _CHUNK):
            sl = pl.ds(row0 + cchunk * _CHUNK, _CHUNK)
            chunks.append((src.at[sl], dst.at[sl]))

    n = len(chunks)
    in_h = [None] * n
    out_h = [None] * n
    for k in range(min(_NBUF, n)):
        in_h[k] = pltpu.async_copy(chunks[k][0], bufs[k % _NBUF],
                                   sem_in[k % _NBUF])
    for k in range(n):
        in_h[k].wait()
        out_h[k] = pltpu.async_copy(bufs[k % _NBUF], chunks[k][1],
                                    sem_out[k % _NBUF])
        kn = k + _NBUF
        if kn < n:
            out_h[k].wait()
            in_h[kn] = pltpu.async_copy(chunks[kn][0], bufs[kn % _NBUF],
                                        sem_in[kn % _NBUF])
    for k in range(max(0, n - _NBUF), n):
        out_h[k].wait()

    plsc.subcore_barrier()

    h_a = pltpu.async_copy(zbuf, dst_a.at[idxb.at[0]], sem_sc)
    h_b = pltpu.async_copy(zbuf, dst_b.at[idxb.at[0]], sem_sc)
    h_a.wait()
    h_b.wait()


def _tc_mask_kernel(j_ref, dgb_ref, prb_ref, dgp_ref, prp_ref,
                    age_ref, gen_ref,
                    o_dgb, o_prb, o_dgp, o_prp, o_age, o_gen):
    j = j_ref[...]
    z = jnp.float32(0.0)
    keep0 = (j != 0).astype(jnp.float32)[:, :, None]
    keep1 = (j != 1).astype(jnp.float32)[:, :, None]
    o_dgb[...] = dgb_ref[...] * keep0
    o_dgp[...] = dgp_ref[...] * keep0
    o_prb[...] = prb_ref[...] * keep1
    o_prp[...] = prp_ref[...] * keep1
    o_age[...] = jnp.where(j != 3, age_ref[...], z)
    o_gen[...] = jnp.where(j != 4, gen_ref[...], z)


def kernel(x_dg_bin, x_prod_bin, x_odb_bin, x_dg_pe, x_prod_pe, x_odb_pe,
           x_age, x_gender):
    B, D = x_dg_bin.shape
    R = B // 128
    j = _make_field_idx(B)
    idx = _scatter_indices(j, B, _SC_FIELD)
    zeros = jnp.zeros((_CAP, D), jnp.float32)

    big_t = jax.ShapeDtypeStruct((B, D), jnp.float32)
    mesh = plsc.VectorSubcoreMesh(core_axis_name="c", subcore_axis_name="s")
    sc_fn = pl.kernel(
        _sc_mask_copy,
        out_type=[big_t] * 2,
        mesh=mesh,
        scratch_types=(
            [pltpu.VMEM((_CHUNK, D), jnp.float32)] * _NBUF
            + [pltpu.VMEM((_CAP, D), jnp.float32),
               pltpu.VMEM((1, _CAP), jnp.int32)]
            + [pltpu.SemaphoreType.DMA] * (2 * _NBUF + 1)
        ),
    )
    o_odb, o_odp = sc_fn(x_odb_bin, x_odb_pe, idx, zeros)

    jp = j.reshape(R, 128)
    bm = 32
    grid = (R // bm,)
    big3 = [x.reshape(R, 128, D) for x in
            (x_dg_bin, x_prod_bin, x_dg_pe, x_prod_pe)]
    bigs = pl.BlockSpec((bm, 128, D), lambda i: (i, 0, 0))
    vec = pl.BlockSpec((bm, 128), lambda i: (i, 0))
    big3_t = jax.ShapeDtypeStruct((R, 128, D), jnp.float32)
    vec_t = jax.ShapeDtypeStruct((R, 128), jnp.float32)

    o_dgb, o_prb, o_dgp, o_prp, o_age, o_gen = pl.pallas_call(
        _tc_mask_kernel,
        grid=grid,
        in_specs=[vec, bigs, bigs, bigs, bigs, vec, vec],
        out_specs=[bigs, bigs, bigs, bigs, vec, vec],
        out_shape=[big3_t, big3_t, big3_t, big3_t, vec_t, vec_t],
    )(jp, *big3, x_age.reshape(R, 128), x_gender.reshape(R, 128))

    return (o_dgb.reshape(B, D), o_prb.reshape(B, D), o_odb,
            o_dgp.reshape(B, D), o_prp.reshape(B, D), o_odp,
            o_age.reshape(B), o_gen.reshape(B))


# speedup vs baseline: 4.5117x; 2.4216x over previous
import jax
import jax.numpy as jnp
from jax import lax
from jax.experimental import pallas as pl
from jax.experimental.pallas import tpu as pltpu
from jax.experimental.pallas import tpu_sc as plsc

_MASK_PCT = 0.8
_STRIPE = 512


def _make_field_idx(bs: int):
    n_masked = int(_MASK_PCT * bs)
    jkey = jax.random.key(42)
    j = jax.random.randint(jkey, (n_masked,), 0, 5, dtype=jnp.int32)
    return jnp.concatenate([j, -jnp.ones((bs - n_masked,), dtype=jnp.int32)])


def _sc_vec_mask(j_hbm, age_hbm, gen_hbm, o_age, o_gen,
                 jv, av, gv, oav, ogv):
    c = lax.axis_index("c")
    s = lax.axis_index("s")
    row0 = (c * 16 + s) * _STRIPE

    sl = pl.ds(row0, _STRIPE)
    pltpu.sync_copy(j_hbm.at[sl], jv)
    pltpu.sync_copy(age_hbm.at[sl], av)
    pltpu.sync_copy(gen_hbm.at[sl], gv)

    def body(i, carry):
        v = pl.ds(i * 16, 16)
        jj = jv[v]
        z = jnp.zeros((16,), jnp.float32)
        oav[v] = jnp.where(jj != 3, av[v], z)
        ogv[v] = jnp.where(jj != 4, gv[v], z)
        return carry

    lax.fori_loop(0, _STRIPE // 16, body, 0)

    pltpu.sync_copy(oav, o_age.at[sl])
    pltpu.sync_copy(ogv, o_gen.at[sl])


def _tc_mask_kernel(j_ref, dgb_ref, prb_ref, odb_ref, dgp_ref, prp_ref,
                    odp_ref, o_dgb, o_prb, o_odb, o_dgp, o_prp, o_odp):
    j = j_ref[...]
    keep0 = (j != 0).astype(jnp.float32)[:, :, None]
    keep1 = (j != 1).astype(jnp.float32)[:, :, None]
    keep2 = (j != 2).astype(jnp.float32)[:, :, None]
    o_dgb[...] = dgb_ref[...] * keep0
    o_dgp[...] = dgp_ref[...] * keep0
    o_prb[...] = prb_ref[...] * keep1
    o_prp[...] = prp_ref[...] * keep1
    o_odb[...] = odb_ref[...] * keep2
    o_odp[...] = odp_ref[...] * keep2


def kernel(x_dg_bin, x_prod_bin, x_odb_bin, x_dg_pe, x_prod_pe, x_odb_pe,
           x_age, x_gender):
    B, D = x_dg_bin.shape
    R = B // 128
    j = _make_field_idx(B)

    vecs_t = jax.ShapeDtypeStruct((B,), jnp.float32)
    mesh = plsc.VectorSubcoreMesh(core_axis_name="c", subcore_axis_name="s")
    sc_fn = pl.kernel(
        _sc_vec_mask,
        out_type=[vecs_t, vecs_t],
        mesh=mesh,
        scratch_types=[
            pltpu.VMEM((_STRIPE,), jnp.int32),
            pltpu.VMEM((_STRIPE,), jnp.float32),
            pltpu.VMEM((_STRIPE,), jnp.float32),
            pltpu.VMEM((_STRIPE,), jnp.float32),
            pltpu.VMEM((_STRIPE,), jnp.float32),
        ],
    )
    o_age, o_gen = sc_fn(j, x_age, x_gender)

    jp = j.reshape(R, 128)
    bm = 32
    grid = (R // bm,)
    big3 = [x.reshape(R, 128, D) for x in
            (x_dg_bin, x_prod_bin, x_odb_bin, x_dg_pe, x_prod_pe, x_odb_pe)]
    bigs = pl.BlockSpec((bm, 128, D), lambda i: (i, 0, 0))
    vec = pl.BlockSpec((bm, 128), lambda i: (i, 0))
    big3_t = jax.ShapeDtypeStruct((R, 128, D), jnp.float32)

    o_dgb, o_prb, o_odb, o_dgp, o_prp, o_odp = pl.pallas_call(
        _tc_mask_kernel,
        grid=grid,
        in_specs=[vec] + [bigs] * 6,
        out_specs=[bigs] * 6,
        out_shape=[big3_t] * 6,
    )(jp, *big3)

    return (o_dgb.reshape(B, D), o_prb.reshape(B, D), o_odb.reshape(B, D),
            o_dgp.reshape(B, D), o_prp.reshape(B, D), o_odp.reshape(B, D),
            o_age, o_gen)
